# baseline (speedup 1.0000x reference)
import jax
import jax.numpy as jnp
from jax import lax
from jax.experimental import pallas as pl
from jax.experimental.pallas import tpu as pltpu
from jax.experimental.pallas import tpu_sc as plsc

BATCH = 16384
EMBED_K = 64

_NUM_CORES = 2
_NUM_SUBCORES = 16
_NUM_WORKERS = _NUM_CORES * _NUM_SUBCORES
_BPW = BATCH // _NUM_WORKERS


def _sc_gather_body(ui_hbm, vi_hbm, w_hbm, h_hbm, outu_hbm, outv_hbm,
                    idx_u, idx_v, rows_u, rows_v, sem_u, sem_v):
    wid = lax.axis_index("s") * _NUM_CORES + lax.axis_index("c")
    base = wid * _BPW
    pltpu.sync_copy(ui_hbm.at[pl.ds(base, _BPW)], idx_u)
    pltpu.sync_copy(vi_hbm.at[pl.ds(base, _BPW)], idx_v)
    cu = pltpu.async_copy(w_hbm.at[idx_u], rows_u, sem_u)
    cv = pltpu.async_copy(h_hbm.at[idx_v], rows_v, sem_v)
    cu.wait()
    pltpu.sync_copy(rows_u, outu_hbm.at[pl.ds(base, _BPW)])
    cv.wait()
    pltpu.sync_copy(rows_v, outv_hbm.at[pl.ds(base, _BPW)])


def _sc_gather(ui, vi, W, H):
    mesh = plsc.VectorSubcoreMesh(core_axis_name="c", subcore_axis_name="s")
    run = pl.kernel(
        _sc_gather_body,
        mesh=mesh,
        out_type=[
            jax.ShapeDtypeStruct((BATCH, EMBED_K), jnp.bfloat16),
            jax.ShapeDtypeStruct((BATCH, EMBED_K), jnp.bfloat16),
        ],
        scratch_types=[
            pltpu.VMEM((_BPW,), jnp.int32),
            pltpu.VMEM((_BPW,), jnp.int32),
            pltpu.VMEM((_BPW, EMBED_K), jnp.bfloat16),
            pltpu.VMEM((_BPW, EMBED_K), jnp.bfloat16),
            pltpu.SemaphoreType.DMA,
            pltpu.SemaphoreType.DMA,
        ],
        compiler_params=pltpu.CompilerParams(use_tc_tiling_on_sc=False),
    )
    return run(ui, vi, W, H)


_BLK = 2048


def _mlp_body(u_ref, v_ref, w1a_ref, w1b_ref, b1_ref, w2_ref, b2_ref,
              w3_ref, b3_ref, out_ref):
    z1 = (
        jnp.dot(u_ref[...], w1a_ref[...], preferred_element_type=jnp.float32)
        + jnp.dot(v_ref[...], w1b_ref[...], preferred_element_type=jnp.float32)
        + b1_ref[...]
    )
    h1 = jnp.maximum(z1, 0.0)
    z2 = jnp.dot(h1, w2_ref[...], preferred_element_type=jnp.float32) + b2_ref[...]
    h2 = jnp.maximum(z2, 0.0)
    out_ref[...] = jnp.sum(h2 * w3_ref[...], axis=1) + b3_ref[0, 0]


def _tc_mlp(U, V, W1, b1, W2, b2, W3, b3):
    w1a = W1[:EMBED_K].astype(jnp.bfloat16)
    w1b = W1[EMBED_K:].astype(jnp.bfloat16)
    b1r = b1.reshape(1, EMBED_K)
    b2r = b2.reshape(1, EMBED_K)
    w3r = W3.reshape(1, EMBED_K)
    b3r = b3.reshape(1, 1)
    grid = BATCH // _BLK
    full = pl.BlockSpec((EMBED_K, EMBED_K), lambda i: (0, 0))
    row = pl.BlockSpec((1, EMBED_K), lambda i: (0, 0))
    return pl.pallas_call(
        _mlp_body,
        grid=(grid,),
        in_specs=[
            pl.BlockSpec((_BLK, EMBED_K), lambda i: (i, 0)),
            pl.BlockSpec((_BLK, EMBED_K), lambda i: (i, 0)),
            full, full, row, full, row, row,
            pl.BlockSpec((1, 1), lambda i: (0, 0)),
        ],
        out_specs=pl.BlockSpec((_BLK,), lambda i: (i,)),
        out_shape=jax.ShapeDtypeStruct((BATCH,), jnp.float32),
    )(U, V, w1a, w1b, b1r, W2, b2r, w3r, b3r)


def kernel(x, W, H, W1, b1, W2, b2, W3, b3):
    ui = x[:, 0].astype(jnp.int32)
    vi = x[:, 1].astype(jnp.int32)
    U, V = _sc_gather(ui, vi, W.astype(jnp.bfloat16), H.astype(jnp.bfloat16))
    return _tc_mlp(U, V, W1, b1, W2, b2, W3, b3)


# speedup vs baseline: 1.2075x; 1.2075x over previous
import jax
import jax.numpy as jnp
from jax import lax
from jax.experimental import pallas as pl
from jax.experimental.pallas import tpu as pltpu
from jax.experimental.pallas import tpu_sc as plsc

BATCH = 16384
EMBED_K = 64

_NUM_CORES = 2
_NUM_SUBCORES = 16
_NUM_WORKERS = _NUM_CORES * _NUM_SUBCORES
_BPW = BATCH // _NUM_WORKERS


def _sc_gather_body(x_hbm, w_hbm, h_hbm, outu_hbm, outv_hbm,
                    xbuf, idx_u, idx_v, rows_u, rows_v, sem_u, sem_v):
    wid = lax.axis_index("s") * _NUM_CORES + lax.axis_index("c")
    base = wid * _BPW
    lanes = lax.iota(jnp.int32, 16)
    pltpu.sync_copy(x_hbm.at[pl.ds(base, _BPW), :], xbuf)

    def deint(i, acc):
        pos = i * 16
        u = plsc.load_gather(xbuf, [pos + lanes, jnp.zeros((16,), jnp.int32)])
        v = plsc.load_gather(xbuf, [pos + lanes, jnp.ones((16,), jnp.int32)])
        idx_u[pl.ds(pos, 16)] = u
        idx_v[pl.ds(pos, 16)] = v
        return acc

    lax.fori_loop(0, _BPW // 16, deint, jnp.int32(0))
    cu = pltpu.async_copy(w_hbm.at[idx_u], rows_u, sem_u)
    cv = pltpu.async_copy(h_hbm.at[idx_v], rows_v, sem_v)
    cu.wait()
    pltpu.sync_copy(rows_u, outu_hbm.at[pl.ds(base, _BPW)])
    cv.wait()
    pltpu.sync_copy(rows_v, outv_hbm.at[pl.ds(base, _BPW)])


def _sc_gather(x, W, H):
    mesh = plsc.VectorSubcoreMesh(core_axis_name="c", subcore_axis_name="s")
    run = pl.kernel(
        _sc_gather_body,
        mesh=mesh,
        out_type=[
            jax.ShapeDtypeStruct((BATCH, EMBED_K), jnp.float32),
            jax.ShapeDtypeStruct((BATCH, EMBED_K), jnp.float32),
        ],
        scratch_types=[
            pltpu.VMEM((_BPW, 2), jnp.int32),
            pltpu.VMEM((_BPW,), jnp.int32),
            pltpu.VMEM((_BPW,), jnp.int32),
            pltpu.VMEM((_BPW, EMBED_K), jnp.float32),
            pltpu.VMEM((_BPW, EMBED_K), jnp.float32),
            pltpu.SemaphoreType.DMA,
            pltpu.SemaphoreType.DMA,
        ],
        compiler_params=pltpu.CompilerParams(
            use_tc_tiling_on_sc=False, needs_layout_passes=False),
    )
    return run(x, W, H)


_BLK = 2048


def _mlp_body(u_ref, v_ref, w1a_ref, w1b_ref, b1_ref, w2_ref, b2_ref,
              w3_ref, b3_ref, out_ref):
    z1 = (
        jnp.dot(u_ref[...], w1a_ref[...], preferred_element_type=jnp.float32)
        + jnp.dot(v_ref[...], w1b_ref[...], preferred_element_type=jnp.float32)
        + b1_ref[...]
    )
    h1 = jnp.maximum(z1, 0.0)
    z2 = jnp.dot(h1, w2_ref[...], preferred_element_type=jnp.float32) + b2_ref[...]
    h2 = jnp.maximum(z2, 0.0)
    out_ref[...] = jnp.sum(h2 * w3_ref[...], axis=1) + b3_ref[0, 0]


def _tc_mlp(U, V, W1, b1, W2, b2, W3, b3):
    w1a = W1[:EMBED_K]
    w1b = W1[EMBED_K:]
    b1r = b1.reshape(1, EMBED_K)
    b2r = b2.reshape(1, EMBED_K)
    w3r = W3.reshape(1, EMBED_K)
    b3r = b3.reshape(1, 1)
    grid = BATCH // _BLK
    full = pl.BlockSpec((EMBED_K, EMBED_K), lambda i: (0, 0))
    row = pl.BlockSpec((1, EMBED_K), lambda i: (0, 0))
    return pl.pallas_call(
        _mlp_body,
        grid=(grid,),
        in_specs=[
            pl.BlockSpec((_BLK, EMBED_K), lambda i: (i, 0)),
            pl.BlockSpec((_BLK, EMBED_K), lambda i: (i, 0)),
            full, full, row, full, row, row,
            pl.BlockSpec((1, 1), lambda i: (0, 0)),
        ],
        out_specs=pl.BlockSpec((_BLK,), lambda i: (i,)),
        out_shape=jax.ShapeDtypeStruct((BATCH,), jnp.float32),
    )(U, V, w1a, w1b, b1r, W2, b2r, w3r, b3r)


def kernel(x, W, H, W1, b1, W2, b2, W3, b3):
    U, V = _sc_gather(x.astype(jnp.int32), W, H)
    return _tc_mlp(U, V, W1, b1, W2, b2, W3, b3)


# speedup vs baseline: 1.2752x; 1.0561x over previous
import jax
import jax.numpy as jnp
from jax import lax
from jax.experimental import pallas as pl
from jax.experimental.pallas import tpu as pltpu
from jax.experimental.pallas import tpu_sc as plsc

BATCH = 16384
EMBED_K = 64

_NUM_CORES = 2
_NUM_SUBCORES = 16
_NUM_WORKERS = _NUM_CORES * _NUM_SUBCORES
_BPW = BATCH // _NUM_WORKERS


def _sc_gather_body(ui_hbm, vi_hbm, w_hbm, h_hbm, outu_hbm, outv_hbm,
                    idx_u, idx_v, rows_u, rows_v, sem_u, sem_v):
    wid = lax.axis_index("s") * _NUM_CORES + lax.axis_index("c")
    base = wid * _BPW
    pltpu.sync_copy(ui_hbm.at[pl.ds(base, _BPW)], idx_u)
    pltpu.sync_copy(vi_hbm.at[pl.ds(base, _BPW)], idx_v)
    cu = pltpu.async_copy(w_hbm.at[idx_u], rows_u, sem_u)
    cv = pltpu.async_copy(h_hbm.at[idx_v], rows_v, sem_v)
    cu.wait()
    pltpu.sync_copy(rows_u, outu_hbm.at[pl.ds(base, _BPW)])
    cv.wait()
    pltpu.sync_copy(rows_v, outv_hbm.at[pl.ds(base, _BPW)])


def _sc_gather(ui, vi, W, H):
    mesh = plsc.VectorSubcoreMesh(core_axis_name="c", subcore_axis_name="s")
    run = pl.kernel(
        _sc_gather_body,
        mesh=mesh,
        out_type=[
            jax.ShapeDtypeStruct((BATCH, EMBED_K), jnp.float32),
            jax.ShapeDtypeStruct((BATCH, EMBED_K), jnp.float32),
        ],
        scratch_types=[
            pltpu.VMEM((_BPW,), jnp.int32),
            pltpu.VMEM((_BPW,), jnp.int32),
            pltpu.VMEM((_BPW, EMBED_K), jnp.float32),
            pltpu.VMEM((_BPW, EMBED_K), jnp.float32),
            pltpu.SemaphoreType.DMA,
            pltpu.SemaphoreType.DMA,
        ],
        compiler_params=pltpu.CompilerParams(use_tc_tiling_on_sc=False),
    )
    return run(ui, vi, W, H)


_BLK = 2048


def _mlp_body(u_ref, v_ref, w1a_ref, w1b_ref, b1_ref, w2_ref, b2_ref,
              w3_ref, b3_ref, out_ref):
    z1 = (
        jnp.dot(u_ref[...], w1a_ref[...], preferred_element_type=jnp.float32)
        + jnp.dot(v_ref[...], w1b_ref[...], preferred_element_type=jnp.float32)
        + b1_ref[...]
    )
    h1 = jnp.maximum(z1, 0.0)
    z2 = jnp.dot(h1, w2_ref[...], preferred_element_type=jnp.float32) + b2_ref[...]
    h2 = jnp.maximum(z2, 0.0)
    out_ref[...] = jnp.sum(h2 * w3_ref[...], axis=1) + b3_ref[0, 0]


def _tc_mlp(U, V, W1, b1, W2, b2, W3, b3):
    w1a = W1[:EMBED_K]
    w1b = W1[EMBED_K:]
    b1r = b1.reshape(1, EMBED_K)
    b2r = b2.reshape(1, EMBED_K)
    w3r = W3.reshape(1, EMBED_K)
    b3r = b3.reshape(1, 1)
    grid = BATCH // _BLK
    full = pl.BlockSpec((EMBED_K, EMBED_K), lambda i: (0, 0))
    row = pl.BlockSpec((1, EMBED_K), lambda i: (0, 0))
    return pl.pallas_call(
        _mlp_body,
        grid=(grid,),
        in_specs=[
            pl.BlockSpec((_BLK, EMBED_K), lambda i: (i, 0)),
            pl.BlockSpec((_BLK, EMBED_K), lambda i: (i, 0)),
            full, full, row, full, row, row,
            pl.BlockSpec((1, 1), lambda i: (0, 0)),
        ],
        out_specs=pl.BlockSpec((_BLK,), lambda i: (i,)),
        out_shape=jax.ShapeDtypeStruct((BATCH,), jnp.float32),
    )(U, V, w1a, w1b, b1r, W2, b2r, w3r, b3r)


def kernel(x, W, H, W1, b1, W2, b2, W3, b3):
    ui = x[:, 0].astype(jnp.int32)
    vi = x[:, 1].astype(jnp.int32)
    U, V = _sc_gather(ui, vi, W, H)
    return _tc_mlp(U, V, W1, b1, W2, b2, W3, b3)
